# baseline (speedup 1.0000x reference)
import functools

import jax
import jax.numpy as jnp
from jax import lax
from jax.experimental import pallas as pl
from jax.experimental.pallas import tpu as pltpu
from jax.experimental.pallas import tpu_sc as plsc

N = 10000
E = 320000
D = 128

NC = 2
NS = 16
NW = NC * NS
EPW = E // NW
CH = 40
NCHUNK = EPW // CH
RING = 6
NP = 10240
RPT = NP // NS
ZR = 32

_mesh = plsc.VectorSubcoreMesh(
    core_axis_name="c", subcore_axis_name="s", num_cores=NC, num_subcores=NS
)


DCH = 80


def _deg_body(edge_hbm, out_hbm, idx_v, ones_v, zbuf, acc, dsem0, dsem1):
    dsem = (dsem0, dsem1)
    c = lax.axis_index("c")
    s = lax.axis_index("s")
    wid = s * NC + c
    base = wid * EPW

    ones = jnp.ones((16,), jnp.float32)
    zeros = jnp.zeros((16,), jnp.float32)

    def fill_it(i, carry):
        for j in range(8):
            ones_v[i, pl.ds(j * 16, 16)] = ones
        return carry

    lax.fori_loop(0, DCH, fill_it, 0)

    def zero_it(i, carry):
        for j in range(8):
            zbuf[i, pl.ds(j * 16, 16)] = zeros
        return carry

    lax.fori_loop(0, ZR, zero_it, 0)

    for which in (0, 1):
        for k in range(RPT // ZR):
            pltpu.sync_copy(zbuf, acc.at[pl.ds(s * RPT + k * ZR, ZR)])
        plsc.subcore_barrier()

        NDC = EPW // DCH

        def blk(j0, carry):
            for b in range(2):
                j = j0 * 2 + b

                @pl.when(j < NDC)
                def _():
                    @pl.when(j >= 2)
                    def _():
                        pltpu.make_async_copy(ones_v, acc.at[idx_v.at[b]],
                                              dsem[b]).wait()

                    off = base + j * DCH
                    pltpu.sync_copy(edge_hbm.at[pl.ds(which * E + off, DCH)],
                                    idx_v.at[b])
                    pltpu.async_copy(ones_v, acc.at[idx_v.at[b]], dsem[b],
                                     add=True)

            return carry

        lax.fori_loop(0, (NDC + 1) // 2, blk, 0)
        for b in range(2):
            pltpu.make_async_copy(ones_v, acc.at[idx_v.at[b]], dsem[b]).wait()
        plsc.subcore_barrier()

        pltpu.sync_copy(acc.at[pl.ds(s * RPT, RPT)],
                        out_hbm.at[c * 2 + which, pl.ds(s * RPT, RPT)])
        plsc.subcore_barrier()


_deg_kernel = functools.partial(
    pl.kernel,
    out_type=jax.ShapeDtypeStruct((2 * NC, NP, D), jnp.float32),
    mesh=_mesh,
    scratch_types=[
        pltpu.VMEM((2, DCH), jnp.int32),
        pltpu.VMEM((DCH, D), jnp.float32),
        pltpu.VMEM((ZR, D), jnp.float32),
        pltpu.VMEM_SHARED((NP, D), jnp.float32),
        pltpu.SemaphoreType.DMA,
        pltpu.SemaphoreType.DMA,
    ],
)(_deg_body)


def _agg_body(edge_hbm, hs_hbm, out_hbm, sidx, didx, *rest):
    rows = rest[:RING]
    zbuf = rest[RING]
    acc = rest[RING + 1]
    semg = rest[RING + 2:2 * RING + 2]
    sems = rest[2 * RING + 2:]
    c = lax.axis_index("c")
    s = lax.axis_index("s")
    wid = s * NC + c
    base = wid * EPW

    def zero_it(i, carry):
        for j in range(8):
            zbuf[i, pl.ds(j * 16, 16)] = jnp.zeros((16,), jnp.float32)
        return carry

    lax.fori_loop(0, ZR, zero_it, 0)
    for k in range(RPT // ZR):
        pltpu.sync_copy(zbuf, acc.at[pl.ds(s * RPT + k * ZR, ZR)])
    plsc.subcore_barrier()

    def issue(j, b):
        off = base + j * CH
        pltpu.sync_copy(edge_hbm.at[pl.ds(off, CH)], sidx.at[b])
        pltpu.sync_copy(edge_hbm.at[pl.ds(E + off, CH)], didx.at[b])
        pltpu.async_copy(hs_hbm.at[sidx.at[b]], rows[b], semg[b])

    for p in range(RING - 1):
        issue(p, p)

    def outer(j0, carry):
        for b in range(RING):
            j = j0 * RING + b
            b2 = (b + RING - 1) % RING

            @pl.when(j < NCHUNK)
            def _():
                pltpu.make_async_copy(hs_hbm.at[sidx.at[b]], rows[b],
                                      semg[b]).wait()
                pltpu.async_copy(rows[b], acc.at[didx.at[b]], sems[b],
                                 add=True)

                @pl.when(j >= 1)
                def _():
                    pltpu.make_async_copy(rows[b2], acc.at[didx.at[b2]],
                                          sems[b2]).wait()

                @pl.when(j + RING - 1 < NCHUNK)
                def _():
                    issue(j + RING - 1, b2)

        return carry

    lax.fori_loop(0, (NCHUNK + RING - 1) // RING, outer, 0)
    lastb = (NCHUNK - 1) % RING
    pltpu.make_async_copy(rows[lastb], acc.at[didx.at[lastb]],
                          sems[lastb]).wait()
    plsc.subcore_barrier()

    pltpu.sync_copy(acc.at[pl.ds(s * RPT, RPT)], out_hbm.at[c, pl.ds(s * RPT, RPT)])


_agg_kernel = functools.partial(
    pl.kernel,
    out_type=jax.ShapeDtypeStruct((NC, NP, D), jnp.float32),
    mesh=_mesh,
    scratch_types=(
        [
            pltpu.VMEM((RING, CH), jnp.int32),
            pltpu.VMEM((RING, CH), jnp.int32),
        ]
        + [pltpu.VMEM((CH, D), jnp.float32) for _ in range(RING)]
        + [
            pltpu.VMEM((ZR, D), jnp.float32),
            pltpu.VMEM_SHARED((NP, D), jnp.float32),
        ]
        + [pltpu.SemaphoreType.DMA for _ in range(2 * RING)]
    ),
)(_agg_body)


def _agg_call(edge, hs):
    return _agg_kernel(edge, hs)


def _norms_body(hist_ref, out_ref):
    deg = jnp.sum(hist_ref[...], axis=2) * (1.0 / D)
    deg_s = deg[0] + deg[2]
    deg_d = deg[1] + deg[3]
    both = jnp.stack([deg_s, deg_d], axis=0)[:, :N]
    out_ref[...] = lax.rsqrt(jnp.clip(both, 1.0, None))


def _norms_call(hist):
    return pl.pallas_call(
        _norms_body,
        out_shape=jax.ShapeDtypeStruct((2, N), jnp.float32),
    )(hist)


def _scale_body(x_ref, n_ref, out_ref):
    out_ref[...] = x_ref[...] * n_ref[...]


def _scale_call(x, n_col):
    return pl.pallas_call(
        _scale_body,
        out_shape=jax.ShapeDtypeStruct((N, D), jnp.float32),
    )(x, n_col)


BN = 2000


def _layer_body(parts_ref, nd_ref, ns_ref, w_ref, b_ref, out_ref):
    agg = (parts_ref[0] + parts_ref[1]) * nd_ref[...]
    y = jnp.dot(agg, w_ref[...], preferred_element_type=jnp.float32) + b_ref[...]
    h = jnp.where(y >= 0.0, y, 0.01 * y)
    out_ref[...] = h * ns_ref[...]


def _layer_call(parts, nd_col, ns_col, w, b_row):
    return pl.pallas_call(
        _layer_body,
        grid=(N // BN,),
        in_specs=[
            pl.BlockSpec((NC, BN, D), lambda i: (0, i, 0)),
            pl.BlockSpec((BN, 1), lambda i: (i, 0)),
            pl.BlockSpec((BN, 1), lambda i: (i, 0)),
            pl.BlockSpec((D, D), lambda i: (0, 0)),
            pl.BlockSpec((1, D), lambda i: (0, 0)),
        ],
        out_specs=pl.BlockSpec((BN, D), lambda i: (i, 0)),
        out_shape=jax.ShapeDtypeStruct((N, D), jnp.float32),
    )(parts, nd_col, ns_col, w, b_row)


def _mean_body(x_ref, out_ref):
    i = pl.program_id(0)
    part = jnp.sum(x_ref[...], axis=0, keepdims=True) * (1.0 / N)

    @pl.when(i == 0)
    def _():
        out_ref[...] = jnp.zeros_like(out_ref)

    out_ref[...] += part


def _mean_call(x):
    return pl.pallas_call(
        _mean_body,
        grid=(N // BN,),
        in_specs=[pl.BlockSpec((BN, D), lambda i: (i, 0))],
        out_specs=pl.BlockSpec((1, D), lambda i: (0, 0)),
        out_shape=jax.ShapeDtypeStruct((1, D), jnp.float32),
    )(x)


def _p1_body(edge_hbm, feat_hbm, out_hbm, idxv, rows, zbuf, acc16, sem):
    c = lax.axis_index("c")
    s = lax.axis_index("s")
    wid = s * NC + c

    zeros = jnp.zeros((16,), jnp.float32)

    def zero_it(i, carry):
        zbuf[i, pl.ds(0, 16)] = zeros
        return carry

    lax.fori_loop(0, 128, zero_it, 0)
    for k in range(5):
        pltpu.sync_copy(zbuf, acc16.at[pl.ds(s * RPT + k * 128, 128)])
    plsc.subcore_barrier()

    pltpu.sync_copy(edge_hbm.at[pl.ds(wid * 128, 128)], idxv)
    pltpu.async_copy(feat_hbm.at[idxv], rows, sem).wait()
    pltpu.sync_copy(zbuf, acc16.at[idxv], add=True)
    plsc.subcore_barrier()
    pltpu.sync_copy(rows, out_hbm.at[pl.ds(wid * 128, 128)])


_p1_kernel = functools.partial(
    pl.kernel,
    out_type=jax.ShapeDtypeStruct((NW * 128, 128), jnp.float32),
    mesh=_mesh,
    scratch_types=[
        pltpu.VMEM((128,), jnp.int32),
        pltpu.VMEM((128, 128), jnp.float32),
        pltpu.VMEM((128, 16), jnp.float32),
        pltpu.VMEM_SHARED((NP, 16), jnp.float32),
        pltpu.SemaphoreType.DMA,
    ],
)(_p1_body)


@jax.jit
def kernel(n_feat, edge_index, W1, b1, W2, b2, W3, b3):
    edge_flat = edge_index.reshape(2 * E)
    hist = _deg_kernel(edge_flat)
    norms = _norms_call(hist)
    ns_col = norms[0][:, None]
    nd_col = norms[1][:, None]

    hs = _scale_call(n_feat, ns_col)

    Ws = jnp.stack([W1, W2, W3])
    bs = jnp.stack([b1, b2, b3]).reshape(3, 1, D)
    scales = jnp.stack([ns_col, ns_col, jnp.ones_like(ns_col)])

    def step(h, xs):
        w, b, sc = xs
        parts = _agg_call(edge_flat, h)
        return _layer_call(parts, nd_col, sc, w, b), None

    hs3, _ = lax.scan(step, hs, (Ws, bs, scales))
    return _mean_call(hs3)


# speedup vs baseline: 1.3827x; 1.3827x over previous
import functools

import jax
import jax.numpy as jnp
from jax import lax
from jax.experimental import pallas as pl
from jax.experimental.pallas import tpu as pltpu
from jax.experimental.pallas import tpu_sc as plsc

N = 10000
E = 320000
D = 128

NC = 2
NS = 16
NW = NC * NS
EPW = E // NW
CH = 80
NCHUNK = EPW // CH
NP = 10240
RPT = NP // NS
ZR = 32

_mesh = plsc.VectorSubcoreMesh(
    core_axis_name="c", subcore_axis_name="s", num_cores=NC, num_subcores=NS
)


DCH = 80


def _deg_body(edge_hbm, out_hbm, idx_v, ones_v, zbuf, acc, dsem0, dsem1):
    dsem = (dsem0, dsem1)
    c = lax.axis_index("c")
    s = lax.axis_index("s")
    wid = s * NC + c
    base = wid * EPW

    ones = jnp.ones((16,), jnp.float32)
    zeros = jnp.zeros((16,), jnp.float32)

    def fill_it(i, carry):
        for j in range(8):
            ones_v[i, pl.ds(j * 16, 16)] = ones
        return carry

    lax.fori_loop(0, DCH, fill_it, 0)

    def zero_it(i, carry):
        for j in range(8):
            zbuf[i, pl.ds(j * 16, 16)] = zeros
        return carry

    lax.fori_loop(0, ZR, zero_it, 0)

    for which in (0, 1):
        for k in range(RPT // ZR):
            pltpu.sync_copy(zbuf, acc.at[pl.ds(s * RPT + k * ZR, ZR)])
        plsc.subcore_barrier()

        NDC = EPW // DCH

        def blk(j0, carry):
            for b in range(2):
                j = j0 * 2 + b

                @pl.when(j < NDC)
                def _():
                    @pl.when(j >= 2)
                    def _():
                        pltpu.make_async_copy(ones_v, acc.at[idx_v.at[b]],
                                              dsem[b]).wait()

                    off = base + j * DCH
                    pltpu.sync_copy(edge_hbm.at[pl.ds(which * E + off, DCH)],
                                    idx_v.at[b])
                    pltpu.async_copy(ones_v, acc.at[idx_v.at[b]], dsem[b],
                                     add=True)

            return carry

        lax.fori_loop(0, (NDC + 1) // 2, blk, 0)
        for b in range(2):
            pltpu.make_async_copy(ones_v, acc.at[idx_v.at[b]], dsem[b]).wait()
        plsc.subcore_barrier()

        pltpu.sync_copy(acc.at[pl.ds(s * RPT, RPT)],
                        out_hbm.at[c * 2 + which, pl.ds(s * RPT, RPT)])
        plsc.subcore_barrier()


_deg_kernel = functools.partial(
    pl.kernel,
    out_type=jax.ShapeDtypeStruct((2 * NC, NP, D), jnp.float32),
    mesh=_mesh,
    scratch_types=[
        pltpu.VMEM((2, DCH), jnp.int32),
        pltpu.VMEM((DCH, D), jnp.float32),
        pltpu.VMEM((ZR, D), jnp.float32),
        pltpu.VMEM_SHARED((NP, D), jnp.float32),
        pltpu.SemaphoreType.DMA,
        pltpu.SemaphoreType.DMA,
    ],
)(_deg_body)


def _agg_body(edge_hbm, hs_hbm, out_hbm, sidx, didx, rows0, rows1, rows2,
              zbuf, acc, semg0, semg1, semg2, sems0, sems1, sems2):
    c = lax.axis_index("c")
    s = lax.axis_index("s")
    wid = s * NC + c
    base = wid * EPW
    rows = (rows0, rows1, rows2)
    semg = (semg0, semg1, semg2)
    sems = (sems0, sems1, sems2)

    def zero_it(i, carry):
        for j in range(8):
            zbuf[i, pl.ds(j * 16, 16)] = jnp.zeros((16,), jnp.float32)
        return carry

    lax.fori_loop(0, ZR, zero_it, 0)
    for k in range(RPT // ZR):
        pltpu.sync_copy(zbuf, acc.at[pl.ds(s * RPT + k * ZR, ZR)])
    plsc.subcore_barrier()

    def issue(j, b):
        off = base + j * CH
        pltpu.sync_copy(edge_hbm.at[pl.ds(off, CH)], sidx.at[b])
        pltpu.sync_copy(edge_hbm.at[pl.ds(E + off, CH)], didx.at[b])
        pltpu.async_copy(hs_hbm.at[sidx.at[b]], rows[b], semg[b])

    issue(0, 0)
    issue(1, 1)

    def outer(j0, carry):
        for b in range(3):
            j = j0 * 3 + b
            b2 = (b + 2) % 3

            @pl.when(j < NCHUNK)
            def _():
                pltpu.make_async_copy(hs_hbm.at[sidx.at[b]], rows[b],
                                      semg[b]).wait()
                pltpu.async_copy(rows[b], acc.at[didx.at[b]], sems[b],
                                 add=True)

                @pl.when(j >= 1)
                def _():
                    pltpu.make_async_copy(rows[b2], acc.at[didx.at[b2]],
                                          sems[b2]).wait()

                @pl.when(j + 2 < NCHUNK)
                def _():
                    issue(j + 2, b2)

        return carry

    lax.fori_loop(0, (NCHUNK + 2) // 3, outer, 0)
    lastb = (NCHUNK - 1) % 3
    pltpu.make_async_copy(rows[lastb], acc.at[didx.at[lastb]],
                          sems[lastb]).wait()
    plsc.subcore_barrier()

    pltpu.sync_copy(acc.at[pl.ds(s * RPT, RPT)], out_hbm.at[c, pl.ds(s * RPT, RPT)])


_agg_kernel = functools.partial(
    pl.kernel,
    out_type=jax.ShapeDtypeStruct((NC, NP, D), jnp.float32),
    mesh=_mesh,
    scratch_types=[
        pltpu.VMEM((3, CH), jnp.int32),
        pltpu.VMEM((3, CH), jnp.int32),
        pltpu.VMEM((CH, D), jnp.float32),
        pltpu.VMEM((CH, D), jnp.float32),
        pltpu.VMEM((CH, D), jnp.float32),
        pltpu.VMEM((ZR, D), jnp.float32),
        pltpu.VMEM_SHARED((NP, D), jnp.float32),
        pltpu.SemaphoreType.DMA,
        pltpu.SemaphoreType.DMA,
        pltpu.SemaphoreType.DMA,
        pltpu.SemaphoreType.DMA,
        pltpu.SemaphoreType.DMA,
        pltpu.SemaphoreType.DMA,
    ],
)(_agg_body)


def _agg_call(edge, hs):
    return _agg_kernel(edge, hs)


def _norms_body(hist_ref, out_ref):
    deg = jnp.sum(hist_ref[...], axis=2) * (1.0 / D)
    deg_s = deg[0] + deg[2]
    deg_d = deg[1] + deg[3]
    both = jnp.stack([deg_s, deg_d], axis=0)[:, :N]
    out_ref[...] = lax.rsqrt(jnp.clip(both, 1.0, None))


def _norms_call(hist):
    return pl.pallas_call(
        _norms_body,
        out_shape=jax.ShapeDtypeStruct((2, N), jnp.float32),
    )(hist)


def _scale_body(x_ref, n_ref, out_ref):
    out_ref[...] = x_ref[...] * n_ref[...]


def _scale_call(x, n_col):
    return pl.pallas_call(
        _scale_body,
        out_shape=jax.ShapeDtypeStruct((N, D), jnp.float32),
    )(x, n_col)


BN = 2000


def _layer_body(parts_ref, nd_ref, ns_ref, w_ref, b_ref, out_ref):
    agg = (parts_ref[0] + parts_ref[1]) * nd_ref[...]
    y = jnp.dot(agg, w_ref[...], preferred_element_type=jnp.float32) + b_ref[...]
    h = jnp.where(y >= 0.0, y, 0.01 * y)
    out_ref[...] = h * ns_ref[...]


def _layer_call(parts, nd_col, ns_col, w, b_row):
    return pl.pallas_call(
        _layer_body,
        grid=(N // BN,),
        in_specs=[
            pl.BlockSpec((NC, BN, D), lambda i: (0, i, 0)),
            pl.BlockSpec((BN, 1), lambda i: (i, 0)),
            pl.BlockSpec((BN, 1), lambda i: (i, 0)),
            pl.BlockSpec((D, D), lambda i: (0, 0)),
            pl.BlockSpec((1, D), lambda i: (0, 0)),
        ],
        out_specs=pl.BlockSpec((BN, D), lambda i: (i, 0)),
        out_shape=jax.ShapeDtypeStruct((N, D), jnp.float32),
    )(parts, nd_col, ns_col, w, b_row)


def _mean_body(x_ref, out_ref):
    i = pl.program_id(0)
    part = jnp.sum(x_ref[...], axis=0, keepdims=True) * (1.0 / N)

    @pl.when(i == 0)
    def _():
        out_ref[...] = jnp.zeros_like(out_ref)

    out_ref[...] += part


def _mean_call(x):
    return pl.pallas_call(
        _mean_body,
        grid=(N // BN,),
        in_specs=[pl.BlockSpec((BN, D), lambda i: (i, 0))],
        out_specs=pl.BlockSpec((1, D), lambda i: (0, 0)),
        out_shape=jax.ShapeDtypeStruct((1, D), jnp.float32),
    )(x)


@jax.jit
def kernel(n_feat, edge_index, W1, b1, W2, b2, W3, b3):
    edge_flat = edge_index.reshape(2 * E)
    hist = _deg_kernel(edge_flat)
    norms = _norms_call(hist)
    ns_col = norms[0][:, None]
    nd_col = norms[1][:, None]

    hs = _scale_call(n_feat, ns_col)

    Ws = jnp.stack([W1, W2, W3])
    bs = jnp.stack([b1, b2, b3]).reshape(3, 1, D)
    scales = jnp.stack([ns_col, ns_col, jnp.ones_like(ns_col)])

    def step(h, xs):
        w, b, sc = xs
        parts = _agg_call(edge_flat, h)
        return _layer_call(parts, nd_col, sc, w, b), None

    hs3, _ = lax.scan(step, hs, (Ws, bs, scales))
    return _mean_call(hs3)
